# baseline (speedup 1.0000x reference)
import functools

import jax
import jax.numpy as jnp
import numpy as np
from jax import lax
from jax.experimental import pallas as pl
from jax.experimental.pallas import tpu as pltpu
from jax.experimental.pallas import tpu_sc as plsc

NUM_SENSORS = 8192
TOP_K = 128
FEAT = 2048
EPS = 1e-6
RB = 1024
NSTEPS = NUM_SENSORS // RB


_ROTATIONS = ((13, 15, 26, 6), (17, 29, 16, 24))


def _rotl(x, r):
    return (x << np.uint32(r)) | (x >> np.uint32(32 - r))


def _threefry2x32(k0, k1, x0, x1):
    ks = (np.uint32(k0), np.uint32(k1),
          np.uint32(k0) ^ np.uint32(k1) ^ np.uint32(0x1BD11BDA))
    x0 = x0 + ks[0]
    x1 = x1 + ks[1]
    for i in range(5):
        for r in _ROTATIONS[i % 2]:
            x0 = x0 + x1
            x1 = _rotl(x1, r)
            x1 = x0 ^ x1
        x0 = x0 + ks[(i + 1) % 3]
        x1 = x1 + ks[(i + 2) % 3] + np.uint32(i + 1)
    return x0, x1


def _np_fold_in(k0, k1, data):
    a, b = _threefry2x32(k0, k1,
                         np.uint32(data >> 32), np.uint32(data & 0xFFFFFFFF))
    return int(a), int(b)


def _np_uniform(k0, k1, shape, minval=0.0, maxval=1.0):
    i = np.arange(int(np.prod(shape)), dtype=np.uint64)
    hi = (i >> np.uint64(32)).astype(np.uint32)
    lo = (i & np.uint64(0xFFFFFFFF)).astype(np.uint32)
    x0, x1 = _threefry2x32(k0, k1, hi, lo)
    bits = x0 ^ x1
    floats = ((bits >> np.uint32(9)) | np.uint32(0x3F800000)).view(np.float32) \
        - np.float32(1.0)
    u = floats * np.float32(maxval - minval) + np.float32(minval)
    return np.maximum(np.float32(minval), u).reshape(shape)


def _rng_consts():
    old = np.seterr(over="ignore")
    try:
        gk = _np_fold_in(0, 0, 1)
        U = _np_uniform(gk[0], gk[1], (NUM_SENSORS, TOP_K))
        e1 = 1.0 / (np.float32(EPS) - np.log(U + np.float32(EPS)))
        ik = _np_fold_in(0, 0, 2)
        tiny = float(np.finfo(np.float32).tiny)
        Ug = _np_uniform(ik[0], ik[1], (TOP_K, NUM_SENSORS), minval=tiny)
        e2t = np.ascontiguousarray((1.0 / (-np.log(Ug))).T)
        return e1.astype(np.float32), e2t.astype(np.float32)
    finally:
        np.seterr(**old)


_E1, _E2T = _rng_consts()


def _tc_body(beta_ref, alpha_ref, e1_ref, x_ref, z_ref, sf_ref):
    i = pl.program_id(0)

    @pl.when(i == 0)
    def _init():
        sf_ref[0:1, :] = jnp.zeros((1, TOP_K), jnp.float32)

    alpha = alpha_ref[...]
    y = 50.0 * alpha
    sp = (y + jnp.log1p(jnp.exp(-y))) * (1.0 / 50.0) + EPS
    u = jnp.exp2(jnp.log2(sp * e1_ref[...]) * (1.0 / beta_ref[0, 0]))
    sf_ref[0:1, :] += jnp.sum(u, axis=0, keepdims=True)

    acc = jax.lax.dot_general(
        u, x_ref[...],
        dimension_numbers=(((0,), (0,)), ((), ())),
        preferred_element_type=jnp.float32)

    @pl.when(i == 0)
    def _z_init():
        z_ref[...] = acc

    @pl.when(i > 0)
    def _z_acc():
        z_ref[...] += acc

    @pl.when(i == NSTEPS - 1)
    def _finalize():
        z_ref[...] *= 1.0 / sf_ref[0:1, :].reshape(TOP_K, 1)


def _tc_call(beta_arr, alpha, e1, X):
    return pl.pallas_call(
        _tc_body,
        grid=(NSTEPS,),
        in_specs=[
            pl.BlockSpec(memory_space=pltpu.SMEM),
            pl.BlockSpec((RB, TOP_K), lambda i: (i, 0)),
            pl.BlockSpec((RB, TOP_K), lambda i: (i, 0)),
            pl.BlockSpec((RB, FEAT), lambda i: (i, 0)),
        ],
        out_specs=pl.BlockSpec((TOP_K, FEAT), lambda i: (0, 0)),
        out_shape=jax.ShapeDtypeStruct((TOP_K, FEAT), jnp.float32),
        scratch_shapes=[pltpu.VMEM((8, TOP_K), jnp.float32)],
        compiler_params=pltpu.CompilerParams(
            dimension_semantics=("arbitrary",)),
    )(beta_arr, alpha, e1, X)


_NW = 32
_RPW = NUM_SENSORS // _NW
_KV = TOP_K // 16
_SLAB = _RPW * TOP_K


_SC_MESH = plsc.VectorSubcoreMesh(core_axis_name="c", subcore_axis_name="s")


def _wid():
    return lax.axis_index("s") * 2 + lax.axis_index("c")


@functools.partial(
    pl.kernel,
    mesh=_SC_MESH,
    out_type=jax.ShapeDtypeStruct((_NW * TOP_K,), jnp.float32),
    scratch_types=[
        pltpu.VMEM((_SLAB,), jnp.float32),
        pltpu.VMEM((TOP_K,), jnp.float32),
    ],
)
def _sc_sums(alpha_hbm, part_hbm, a_v, buf_v):
    wid = _wid()
    base = wid * _SLAB
    pltpu.sync_copy(alpha_hbm.at[pl.ds(base, _SLAB)], a_v)

    def sum_row(r, acc):
        off = r * TOP_K
        return tuple(acc[k] + a_v[pl.ds(off + 16 * k, 16)]
                     for k in range(_KV))

    acc = lax.fori_loop(0, _RPW, sum_row,
                        tuple(jnp.zeros((16,), jnp.float32)
                              for _ in range(_KV)))
    for k in range(_KV):
        buf_v[pl.ds(16 * k, 16)] = acc[k]
    pltpu.sync_copy(buf_v, part_hbm.at[pl.ds(wid * TOP_K, TOP_K)])


@functools.partial(
    pl.kernel,
    mesh=_SC_MESH,
    out_type=[
        jax.ShapeDtypeStruct((NUM_SENSORS * TOP_K,), jnp.float32),
        jax.ShapeDtypeStruct((_NW * TOP_K,), jnp.float32),
        jax.ShapeDtypeStruct((_NW * TOP_K,), jnp.float32),
    ],
    scratch_types=[
        pltpu.VMEM((_SLAB,), jnp.float32),
        pltpu.VMEM((_SLAB,), jnp.float32),
        pltpu.VMEM((_SLAB,), jnp.float32),
        pltpu.VMEM((TOP_K,), jnp.float32),
        pltpu.VMEM((_NW * TOP_K,), jnp.float32),
    ],
)
def _sc_p_idx(alpha_hbm, e2t_hbm, part_hbm,
              p_hbm, maxv_hbm, maxi_hbm, a_v, e_v, p_v, buf_v, part_v):
    wid = _wid()
    base = wid * _SLAB
    pltpu.sync_copy(alpha_hbm.at[pl.ds(base, _SLAB)], a_v)
    pltpu.sync_copy(e2t_hbm.at[pl.ds(base, _SLAB)], e_v)
    pltpu.sync_copy(part_hbm, part_v)

    csum = [jnp.zeros((16,), jnp.float32) for _ in range(_KV)]
    for w in range(_NW):
        for k in range(_KV):
            csum[k] = csum[k] + part_v[pl.ds(w * TOP_K + 16 * k, 16)]
    inv0 = [1.0 / (c + EPS) for c in csum]
    s2 = [inv0[k] / (csum[k] * inv0[k] + EPS) for k in range(_KV)]

    def pv_row(r, carry):
        mx, ag = carry
        off = r * TOP_K
        ridx = (jnp.full((16,), wid * _RPW, jnp.float32)
                + lax.convert_element_type(r, jnp.float32))
        new_mx, new_ag = [], []
        for k in range(_KV):
            sl = pl.ds(off + 16 * k, 16)
            a = a_v[sl]
            p_v[sl] = a * inv0[k]
            val = (a * s2[k] + EPS) * e_v[sl]
            better = val > mx[k]
            new_mx.append(jnp.where(better, val, mx[k]))
            new_ag.append(jnp.where(better, ridx, ag[k]))
        return tuple(new_mx), tuple(new_ag)

    mx0 = tuple(jnp.full((16,), -3.0e38, jnp.float32) for _ in range(_KV))
    ag0 = tuple(jnp.zeros((16,), jnp.float32) for _ in range(_KV))
    mx, ag = lax.fori_loop(0, _RPW, pv_row, (mx0, ag0))

    pltpu.sync_copy(p_v, p_hbm.at[pl.ds(base, _SLAB)])
    for k in range(_KV):
        buf_v[pl.ds(16 * k, 16)] = mx[k]
    pltpu.sync_copy(buf_v, maxv_hbm.at[pl.ds(wid * TOP_K, TOP_K)])
    for k in range(_KV):
        buf_v[pl.ds(16 * k, 16)] = ag[k]
    pltpu.sync_copy(buf_v, maxi_hbm.at[pl.ds(wid * TOP_K, TOP_K)])


def kernel(X, beta, alpha):
    beta_arr = jnp.asarray(beta, jnp.float32).reshape(1, 1)
    alpha = jnp.asarray(alpha)
    Z = _tc_call(beta_arr, alpha, jnp.asarray(_E1), jnp.asarray(X))
    part = _sc_sums(alpha.reshape(-1))
    p_flat, maxv, maxi = _sc_p_idx(alpha.reshape(-1),
                                   jnp.asarray(_E2T).reshape(-1), part)
    mv = maxv.reshape(_NW, TOP_K)
    mi = maxi.reshape(_NW, TOP_K)
    w = jnp.argmax(mv, axis=0)
    idx = jnp.take_along_axis(mi, w[None, :], axis=0)[0].astype(jnp.int32)
    return (Z, idx, p_flat.reshape(NUM_SENSORS, TOP_K))


# speedup vs baseline: 1.9870x; 1.9870x over previous
import jax
import jax.numpy as jnp
import numpy as np
from jax.experimental import pallas as pl
from jax.experimental.pallas import tpu as pltpu

NUM_SENSORS = 8192
TOP_K = 128
FEAT = 2048
EPS = 1e-6
RB = 1024
NSTEPS = NUM_SENSORS // RB


_ROTATIONS = ((13, 15, 26, 6), (17, 29, 16, 24))


def _rotl(x, r):
    return (x << np.uint32(r)) | (x >> np.uint32(32 - r))


def _threefry2x32(k0, k1, x0, x1):
    ks = (np.uint32(k0), np.uint32(k1),
          np.uint32(k0) ^ np.uint32(k1) ^ np.uint32(0x1BD11BDA))
    x0 = x0 + ks[0]
    x1 = x1 + ks[1]
    for i in range(5):
        for r in _ROTATIONS[i % 2]:
            x0 = x0 + x1
            x1 = _rotl(x1, r)
            x1 = x0 ^ x1
        x0 = x0 + ks[(i + 1) % 3]
        x1 = x1 + ks[(i + 2) % 3] + np.uint32(i + 1)
    return x0, x1


def _np_fold_in(k0, k1, data):
    a, b = _threefry2x32(k0, k1,
                         np.uint32(data >> 32), np.uint32(data & 0xFFFFFFFF))
    return int(a), int(b)


def _np_uniform(k0, k1, shape, minval=0.0, maxval=1.0):
    i = np.arange(int(np.prod(shape)), dtype=np.uint64)
    hi = (i >> np.uint64(32)).astype(np.uint32)
    lo = (i & np.uint64(0xFFFFFFFF)).astype(np.uint32)
    x0, x1 = _threefry2x32(k0, k1, hi, lo)
    bits = x0 ^ x1
    floats = ((bits >> np.uint32(9)) | np.uint32(0x3F800000)).view(np.float32) \
        - np.float32(1.0)
    u = floats * np.float32(maxval - minval) + np.float32(minval)
    return np.maximum(np.float32(minval), u).reshape(shape)


def _rng_consts():
    old = np.seterr(over="ignore")
    try:
        gk = _np_fold_in(0, 0, 1)
        U = _np_uniform(gk[0], gk[1], (NUM_SENSORS, TOP_K))
        e1 = 1.0 / (np.float32(EPS) - np.log(U + np.float32(EPS)))
        ik = _np_fold_in(0, 0, 2)
        tiny = float(np.finfo(np.float32).tiny)
        Ug = _np_uniform(ik[0], ik[1], (TOP_K, NUM_SENSORS), minval=tiny)
        e2t = np.ascontiguousarray((1.0 / (-np.log(Ug))).T)
        return e1.astype(np.float32), e2t.astype(np.float32)
    finally:
        np.seterr(**old)


_E1, _E2T = _rng_consts()


def _body(beta_ref, alpha_ref, e1_ref, e2t_ref, x_ref,
          z_ref, p_ref, idx_ref, sf_ref, si_ref):
    i = pl.program_id(0)
    rows = pl.ds(i * RB, RB)

    @pl.when(i == 0)
    def _csum_scales():
        csum = jnp.sum(alpha_ref[...], axis=0, keepdims=True)
        inv0 = 1.0 / (csum + EPS)
        rs = csum * inv0
        sf_ref[0:1, :] = inv0
        sf_ref[1:2, :] = inv0 / (rs + EPS)
        sf_ref[2:3, :] = jnp.zeros((1, TOP_K), jnp.float32)
        sf_ref[3:4, :] = jnp.full((1, TOP_K), -jnp.inf, jnp.float32)
        si_ref[...] = jnp.zeros((1, TOP_K), jnp.int32)

    alpha = alpha_ref[rows, :]

    y = 50.0 * alpha
    sp = (y + jnp.log1p(jnp.exp(-y))) * (1.0 / 50.0) + EPS
    u = jnp.exp2(jnp.log2(sp * e1_ref[...]) * (1.0 / beta_ref[0, 0]))
    sf_ref[2:3, :] += jnp.sum(u, axis=0, keepdims=True)

    p_ref[...] = alpha * sf_ref[0:1, :]

    val = (alpha * sf_ref[1:2, :] + EPS) * e2t_ref[...]
    mx = jnp.max(val, axis=0, keepdims=True)
    iota = jax.lax.broadcasted_iota(jnp.int32, val.shape, 0) + i * RB
    arg = jnp.min(jnp.where(val == mx, iota, NUM_SENSORS), axis=0,
                  keepdims=True)
    better = mx > sf_ref[3:4, :]
    sf_ref[3:4, :] = jnp.where(better, mx, sf_ref[3:4, :])
    si_ref[...] = jnp.where(better, arg, si_ref[...])

    acc = jax.lax.dot_general(
        u, x_ref[...],
        dimension_numbers=(((0,), (0,)), ((), ())),
        preferred_element_type=jnp.float32)

    @pl.when(i == 0)
    def _z_init():
        z_ref[...] = acc

    @pl.when(i > 0)
    def _z_acc():
        z_ref[...] += acc

    @pl.when(i == NSTEPS - 1)
    def _finalize():
        z_ref[...] *= 1.0 / sf_ref[2:3, :].reshape(TOP_K, 1)
        idx_ref[...] = si_ref[...]


def kernel(X, beta, alpha):
    beta_arr = jnp.asarray(beta, jnp.float32).reshape(1, 1)
    Z, p, idx = pl.pallas_call(
        _body,
        grid=(NSTEPS,),
        in_specs=[
            pl.BlockSpec(memory_space=pltpu.SMEM),
            pl.BlockSpec((NUM_SENSORS, TOP_K), lambda i: (0, 0)),
            pl.BlockSpec((RB, TOP_K), lambda i: (i, 0)),
            pl.BlockSpec((RB, TOP_K), lambda i: (i, 0)),
            pl.BlockSpec((RB, FEAT), lambda i: (i, 0)),
        ],
        out_specs=[
            pl.BlockSpec((TOP_K, FEAT), lambda i: (0, 0)),
            pl.BlockSpec((RB, TOP_K), lambda i: (i, 0)),
            pl.BlockSpec((1, TOP_K), lambda i: (0, 0)),
        ],
        out_shape=[
            jax.ShapeDtypeStruct((TOP_K, FEAT), jnp.float32),
            jax.ShapeDtypeStruct((NUM_SENSORS, TOP_K), jnp.float32),
            jax.ShapeDtypeStruct((1, TOP_K), jnp.int32),
        ],
        scratch_shapes=[pltpu.VMEM((8, TOP_K), jnp.float32),
                        pltpu.VMEM((1, TOP_K), jnp.int32)],
        compiler_params=pltpu.CompilerParams(
            dimension_semantics=("arbitrary",)),
    )(beta_arr, jnp.asarray(alpha), jnp.asarray(_E1), jnp.asarray(_E2T),
      jnp.asarray(X))
    return (Z, idx.reshape(TOP_K), p)


# speedup vs baseline: 2.0224x; 1.0178x over previous
import jax
import jax.numpy as jnp
import numpy as np
from jax.experimental import pallas as pl
from jax.experimental.pallas import tpu as pltpu

NUM_SENSORS = 8192
TOP_K = 128
FEAT = 2048
EPS = 1e-6
RB = 1024
NSTEPS = NUM_SENSORS // RB


_ROTATIONS = ((13, 15, 26, 6), (17, 29, 16, 24))


def _rotl(x, r):
    return (x << np.uint32(r)) | (x >> np.uint32(32 - r))


def _threefry2x32(k0, k1, x0, x1):
    ks = (np.uint32(k0), np.uint32(k1),
          np.uint32(k0) ^ np.uint32(k1) ^ np.uint32(0x1BD11BDA))
    x0 = x0 + ks[0]
    x1 = x1 + ks[1]
    for i in range(5):
        for r in _ROTATIONS[i % 2]:
            x0 = x0 + x1
            x1 = _rotl(x1, r)
            x1 = x0 ^ x1
        x0 = x0 + ks[(i + 1) % 3]
        x1 = x1 + ks[(i + 2) % 3] + np.uint32(i + 1)
    return x0, x1


def _np_fold_in(k0, k1, data):
    a, b = _threefry2x32(k0, k1,
                         np.uint32(data >> 32), np.uint32(data & 0xFFFFFFFF))
    return int(a), int(b)


def _np_uniform(k0, k1, shape, minval=0.0, maxval=1.0):
    i = np.arange(int(np.prod(shape)), dtype=np.uint64)
    hi = (i >> np.uint64(32)).astype(np.uint32)
    lo = (i & np.uint64(0xFFFFFFFF)).astype(np.uint32)
    x0, x1 = _threefry2x32(k0, k1, hi, lo)
    bits = x0 ^ x1
    floats = ((bits >> np.uint32(9)) | np.uint32(0x3F800000)).view(np.float32) \
        - np.float32(1.0)
    u = floats * np.float32(maxval - minval) + np.float32(minval)
    return np.maximum(np.float32(minval), u).reshape(shape)


def _rng_consts():
    old = np.seterr(over="ignore")
    try:
        gk = _np_fold_in(0, 0, 1)
        U = _np_uniform(gk[0], gk[1], (NUM_SENSORS, TOP_K))
        e1 = 1.0 / (np.float32(EPS) - np.log(U + np.float32(EPS)))
        ik = _np_fold_in(0, 0, 2)
        tiny = float(np.finfo(np.float32).tiny)
        Ug = _np_uniform(ik[0], ik[1], (TOP_K, NUM_SENSORS), minval=tiny)
        e2t = np.ascontiguousarray((1.0 / (-np.log(Ug))).T)
        import ml_dtypes
        return e1.astype(ml_dtypes.bfloat16), e2t.astype(np.float32)
    finally:
        np.seterr(**old)


_E1, _E2T = _rng_consts()


def _body(beta_ref, alpha_ref, e1_ref, e2t_ref, x_ref,
          z_ref, p_ref, idx_ref, sf_ref, si_ref):
    i = pl.program_id(0)
    rows = pl.ds(i * RB, RB)

    @pl.when(i == 0)
    def _csum_scales():
        csum = jnp.sum(alpha_ref[...], axis=0, keepdims=True)
        inv0 = 1.0 / (csum + EPS)
        rs = csum * inv0
        sf_ref[0:1, :] = inv0
        sf_ref[1:2, :] = inv0 / (rs + EPS)
        sf_ref[2:3, :] = jnp.zeros((1, TOP_K), jnp.float32)
        sf_ref[3:4, :] = jnp.full((1, TOP_K), -jnp.inf, jnp.float32)
        si_ref[...] = jnp.zeros((1, TOP_K), jnp.int32)

    alpha = alpha_ref[rows, :]

    y = 50.0 * alpha
    sp = (y + jnp.log1p(jnp.exp(-y))) * (1.0 / 50.0) + EPS
    u = jnp.exp2(jnp.log2(sp * e1_ref[...].astype(jnp.float32))
                 * (1.0 / beta_ref[0, 0]))
    sf_ref[2:3, :] += jnp.sum(u, axis=0, keepdims=True)

    p_ref[...] = alpha * sf_ref[0:1, :]

    val = (alpha * sf_ref[1:2, :] + EPS) * e2t_ref[...]
    mx = jnp.max(val, axis=0, keepdims=True)
    iota = jax.lax.broadcasted_iota(jnp.int32, val.shape, 0) + i * RB
    arg = jnp.min(jnp.where(val == mx, iota, NUM_SENSORS), axis=0,
                  keepdims=True)
    better = mx > sf_ref[3:4, :]
    sf_ref[3:4, :] = jnp.where(better, mx, sf_ref[3:4, :])
    si_ref[...] = jnp.where(better, arg, si_ref[...])

    acc = jax.lax.dot_general(
        u, x_ref[...],
        dimension_numbers=(((0,), (0,)), ((), ())),
        preferred_element_type=jnp.float32)

    @pl.when(i == 0)
    def _z_init():
        z_ref[...] = acc

    @pl.when(i > 0)
    def _z_acc():
        z_ref[...] += acc

    @pl.when(i == NSTEPS - 1)
    def _finalize():
        z_ref[...] *= 1.0 / sf_ref[2:3, :].reshape(TOP_K, 1)
        idx_ref[...] = si_ref[...]


def kernel(X, beta, alpha):
    beta_arr = jnp.asarray(beta, jnp.float32).reshape(1, 1)
    Z, p, idx = pl.pallas_call(
        _body,
        grid=(NSTEPS,),
        in_specs=[
            pl.BlockSpec(memory_space=pltpu.SMEM),
            pl.BlockSpec((NUM_SENSORS, TOP_K), lambda i: (0, 0)),
            pl.BlockSpec((RB, TOP_K), lambda i: (i, 0)),
            pl.BlockSpec((RB, TOP_K), lambda i: (i, 0)),
            pl.BlockSpec((RB, FEAT), lambda i: (i, 0)),
        ],
        out_specs=[
            pl.BlockSpec((TOP_K, FEAT), lambda i: (0, 0)),
            pl.BlockSpec((RB, TOP_K), lambda i: (i, 0)),
            pl.BlockSpec((1, TOP_K), lambda i: (0, 0)),
        ],
        out_shape=[
            jax.ShapeDtypeStruct((TOP_K, FEAT), jnp.float32),
            jax.ShapeDtypeStruct((NUM_SENSORS, TOP_K), jnp.float32),
            jax.ShapeDtypeStruct((1, TOP_K), jnp.int32),
        ],
        scratch_shapes=[pltpu.VMEM((8, TOP_K), jnp.float32),
                        pltpu.VMEM((1, TOP_K), jnp.int32)],
        compiler_params=pltpu.CompilerParams(
            dimension_semantics=("arbitrary",)),
    )(beta_arr, jnp.asarray(alpha), jnp.asarray(_E1), jnp.asarray(_E2T),
      jnp.asarray(X))
    return (Z, idx.reshape(TOP_K), p)


# speedup vs baseline: 2.0375x; 1.0075x over previous
import jax
import jax.numpy as jnp
import numpy as np
from jax.experimental import pallas as pl
from jax.experimental.pallas import tpu as pltpu

NUM_SENSORS = 8192
TOP_K = 128
FEAT = 2048
EPS = 1e-6
RB = 2048
NSTEPS = NUM_SENSORS // RB


_ROTATIONS = ((13, 15, 26, 6), (17, 29, 16, 24))


def _rotl(x, r):
    return (x << np.uint32(r)) | (x >> np.uint32(32 - r))


def _threefry2x32(k0, k1, x0, x1):
    ks = (np.uint32(k0), np.uint32(k1),
          np.uint32(k0) ^ np.uint32(k1) ^ np.uint32(0x1BD11BDA))
    x0 = x0 + ks[0]
    x1 = x1 + ks[1]
    for i in range(5):
        for r in _ROTATIONS[i % 2]:
            x0 = x0 + x1
            x1 = _rotl(x1, r)
            x1 = x0 ^ x1
        x0 = x0 + ks[(i + 1) % 3]
        x1 = x1 + ks[(i + 2) % 3] + np.uint32(i + 1)
    return x0, x1


def _np_fold_in(k0, k1, data):
    a, b = _threefry2x32(k0, k1,
                         np.uint32(data >> 32), np.uint32(data & 0xFFFFFFFF))
    return int(a), int(b)


def _np_uniform(k0, k1, shape, minval=0.0, maxval=1.0):
    i = np.arange(int(np.prod(shape)), dtype=np.uint64)
    hi = (i >> np.uint64(32)).astype(np.uint32)
    lo = (i & np.uint64(0xFFFFFFFF)).astype(np.uint32)
    x0, x1 = _threefry2x32(k0, k1, hi, lo)
    bits = x0 ^ x1
    floats = ((bits >> np.uint32(9)) | np.uint32(0x3F800000)).view(np.float32) \
        - np.float32(1.0)
    u = floats * np.float32(maxval - minval) + np.float32(minval)
    return np.maximum(np.float32(minval), u).reshape(shape)


def _rng_consts():
    old = np.seterr(over="ignore")
    try:
        gk = _np_fold_in(0, 0, 1)
        U = _np_uniform(gk[0], gk[1], (NUM_SENSORS, TOP_K))
        e1 = 1.0 / (np.float32(EPS) - np.log(U + np.float32(EPS)))
        ik = _np_fold_in(0, 0, 2)
        tiny = float(np.finfo(np.float32).tiny)
        Ug = _np_uniform(ik[0], ik[1], (TOP_K, NUM_SENSORS), minval=tiny)
        e2t = np.ascontiguousarray((1.0 / (-np.log(Ug))).T)
        import ml_dtypes
        return e1.astype(ml_dtypes.bfloat16), e2t.astype(np.float32)
    finally:
        np.seterr(**old)


_E1, _E2T = _rng_consts()


def _body(beta_ref, alpha_ref, e1_ref, e2t_ref, x_ref,
          z_ref, p_ref, idx_ref, sf_ref, si_ref):
    i = pl.program_id(0)
    rows = pl.ds(i * RB, RB)

    @pl.when(i == 0)
    def _csum_scales():
        csum = jnp.sum(alpha_ref[...], axis=0, keepdims=True)
        inv0 = 1.0 / (csum + EPS)
        rs = csum * inv0
        sf_ref[0:1, :] = inv0
        sf_ref[1:2, :] = inv0 / (rs + EPS)
        sf_ref[2:3, :] = jnp.zeros((1, TOP_K), jnp.float32)
        sf_ref[3:4, :] = jnp.full((1, TOP_K), -jnp.inf, jnp.float32)
        si_ref[...] = jnp.zeros((1, TOP_K), jnp.int32)

    alpha = alpha_ref[rows, :]

    y = 50.0 * alpha
    sp = (y + jnp.log1p(jnp.exp(-y))) * (1.0 / 50.0) + EPS
    u = jnp.exp2(jnp.log2(sp * e1_ref[...].astype(jnp.float32))
                 * (1.0 / beta_ref[0, 0]))
    sf_ref[2:3, :] += jnp.sum(u, axis=0, keepdims=True)

    p_ref[...] = alpha * sf_ref[0:1, :]

    val = (alpha * sf_ref[1:2, :] + EPS) * e2t_ref[...]
    mx = jnp.max(val, axis=0, keepdims=True)
    iota = jax.lax.broadcasted_iota(jnp.int32, val.shape, 0) + i * RB
    arg = jnp.min(jnp.where(val == mx, iota, NUM_SENSORS), axis=0,
                  keepdims=True)
    better = mx > sf_ref[3:4, :]
    sf_ref[3:4, :] = jnp.where(better, mx, sf_ref[3:4, :])
    si_ref[...] = jnp.where(better, arg, si_ref[...])

    acc = jax.lax.dot_general(
        u, x_ref[...],
        dimension_numbers=(((0,), (0,)), ((), ())),
        preferred_element_type=jnp.float32)

    @pl.when(i == 0)
    def _z_init():
        z_ref[...] = acc

    @pl.when(i > 0)
    def _z_acc():
        z_ref[...] += acc

    @pl.when(i == NSTEPS - 1)
    def _finalize():
        z_ref[...] *= 1.0 / sf_ref[2:3, :].reshape(TOP_K, 1)
        idx_ref[...] = si_ref[...]


def kernel(X, beta, alpha):
    beta_arr = jnp.asarray(beta, jnp.float32).reshape(1, 1)
    Z, p, idx = pl.pallas_call(
        _body,
        grid=(NSTEPS,),
        in_specs=[
            pl.BlockSpec(memory_space=pltpu.SMEM),
            pl.BlockSpec((NUM_SENSORS, TOP_K), lambda i: (0, 0)),
            pl.BlockSpec((RB, TOP_K), lambda i: (i, 0)),
            pl.BlockSpec((RB, TOP_K), lambda i: (i, 0)),
            pl.BlockSpec((RB, FEAT), lambda i: (i, 0)),
        ],
        out_specs=[
            pl.BlockSpec((TOP_K, FEAT), lambda i: (0, 0)),
            pl.BlockSpec((RB, TOP_K), lambda i: (i, 0)),
            pl.BlockSpec((1, TOP_K), lambda i: (0, 0)),
        ],
        out_shape=[
            jax.ShapeDtypeStruct((TOP_K, FEAT), jnp.float32),
            jax.ShapeDtypeStruct((NUM_SENSORS, TOP_K), jnp.float32),
            jax.ShapeDtypeStruct((1, TOP_K), jnp.int32),
        ],
        scratch_shapes=[pltpu.VMEM((8, TOP_K), jnp.float32),
                        pltpu.VMEM((1, TOP_K), jnp.int32)],
        compiler_params=pltpu.CompilerParams(
            dimension_semantics=("arbitrary",)),
    )(beta_arr, jnp.asarray(alpha), jnp.asarray(_E1), jnp.asarray(_E2T),
      jnp.asarray(X))
    return (Z, idx.reshape(TOP_K), p)
